# Initial kernel scaffold; baseline (speedup 1.0000x reference)
import functools

import jax
import jax.numpy as jnp
from jax import lax
from jax.experimental import pallas as pl
from jax.experimental.pallas import tpu as pltpu
from jax.experimental.pallas import tpu_sc as plsc

B, T, H = 1024, 200, 32
N_STATIC, N_KC, N_KR, N_OBS = 3, 3, 4, 3
STATIC_VOCAB, KC_VOCAB = 100000, 1000
N = B * T

NC, NS = 2, 16
NW = NC * NS

S_PER_W = (B * N_STATIC) // NW
C_PER_W = (N * N_KC) // NW
STREAM_ROWS = 128
STREAMS_PER_CHUNK = 10
CHUNK = STREAM_ROWS * STREAMS_PER_CHUNK
N_CHUNKS = C_PER_W // CHUNK


def _sc_gather(static_flat, kc_flat, st_tab, kc_tab):
    mesh = plsc.VectorSubcoreMesh(
        core_axis_name="c", subcore_axis_name="s", num_cores=NC, num_subcores=NS
    )

    def body(static_hbm, kc_hbm, sttab_hbm, kctab_hbm, sout_hbm, g_hbm,
             sidx, srows, idx_raw, idx2, rows, sem):
        wid = lax.axis_index("s") * NC + lax.axis_index("c")
        iota = lax.iota(jnp.int32, 16)

        sbase = wid * S_PER_W
        pltpu.sync_copy(static_hbm.at[pl.ds(sbase, S_PER_W)], sidx)
        for v in range(S_PER_W // 16):
            k = sbase + v * 16 + iota
            sidx[pl.ds(v * 16, 16)] = (
                sidx[pl.ds(v * 16, 16)] + (k % N_STATIC) * STATIC_VOCAB
            )
        pltpu.async_copy(sttab_hbm.at[sidx], srows, sem).wait()
        pltpu.sync_copy(srows, sout_hbm.at[pl.ds(sbase, S_PER_W)])

        def chunk_body(c, carry):
            base = wid * C_PER_W + c * CHUNK
            pltpu.sync_copy(kc_hbm.at[pl.ds(base, CHUNK)], idx_raw)
            for v in range(CHUNK // 16):
                k = base + v * 16 + iota
                s, col = v // 8, (v % 8) * 16
                idx2[s, pl.ds(col, 16)] = (
                    idx_raw[pl.ds(v * 16, 16)] + (k % N_KC) * KC_VOCAB
                )
            copies = [
                pltpu.async_copy(
                    kctab_hbm.at[idx2.at[s]],
                    rows.at[pl.ds(s * STREAM_ROWS, STREAM_ROWS)],
                    sem,
                )
                for s in range(STREAMS_PER_CHUNK)
            ]
            for cp in copies:
                cp.wait()
            pltpu.sync_copy(rows, g_hbm.at[pl.ds(base, CHUNK)])
            return carry

        lax.fori_loop(0, N_CHUNKS, chunk_body, 0)

    fn = pl.kernel(
        body,
        out_type=[
            jax.ShapeDtypeStruct((B * N_STATIC, H), jnp.float32),
            jax.ShapeDtypeStruct((N * N_KC, H), jnp.float32),
        ],
        mesh=mesh,
        scratch_types=[
            pltpu.VMEM((S_PER_W,), jnp.int32),
            pltpu.VMEM((S_PER_W, H), jnp.float32),
            pltpu.VMEM((CHUNK,), jnp.int32),
            pltpu.VMEM((STREAMS_PER_CHUNK, STREAM_ROWS), jnp.int32),
            pltpu.VMEM((CHUNK, H), jnp.float32),
            pltpu.SemaphoreType.DMA,
        ],
    )
    return fn(static_flat, kc_flat, st_tab, kc_tab)


NB = 2048


def _tc_body(kr_ref, ob_ref, g_ref, mr_ref, mo_ref, p_ref, bk_ref, bo_ref,
             known_ref, obs_ref):
    g = g_ref[...].astype(jnp.bfloat16)
    kr = kr_ref[...].astype(jnp.bfloat16)
    ob = ob_ref[...].astype(jnp.bfloat16)
    dn = (((1,), (0,)), ((), ()))
    known = lax.dot_general(g, p_ref[...], dn, preferred_element_type=jnp.float32)
    known = known + lax.dot_general(
        kr, mr_ref[...], dn, preferred_element_type=jnp.float32
    )
    known_ref[...] = known + bk_ref[...]
    obs = lax.dot_general(ob, mo_ref[...], dn, preferred_element_type=jnp.float32)
    obs_ref[...] = obs + bo_ref[...]


def _tc_assemble(kr2, ob2, g2, mr, mo, p, bk, bo):
    grid = (N // NB,)
    kw = H * (N_KR + N_KC)
    ow = H * N_OBS
    return pl.pallas_call(
        _tc_body,
        grid=grid,
        in_specs=[
            pl.BlockSpec((NB, N_KR), lambda i: (i, 0)),
            pl.BlockSpec((NB, N_OBS), lambda i: (i, 0)),
            pl.BlockSpec((NB, H * N_KC), lambda i: (i, 0)),
            pl.BlockSpec((N_KR, kw), lambda i: (0, 0)),
            pl.BlockSpec((N_OBS, ow), lambda i: (0, 0)),
            pl.BlockSpec((H * N_KC, kw), lambda i: (0, 0)),
            pl.BlockSpec((1, kw), lambda i: (0, 0)),
            pl.BlockSpec((1, ow), lambda i: (0, 0)),
        ],
        out_specs=[
            pl.BlockSpec((NB, kw), lambda i: (i, 0)),
            pl.BlockSpec((NB, ow), lambda i: (i, 0)),
        ],
        out_shape=[
            jax.ShapeDtypeStruct((N, kw), jnp.float32),
            jax.ShapeDtypeStruct((N, ow), jnp.float32),
        ],
    )(kr2, ob2, g2, mr, mo, p, bk, bo)


def kernel(static, known_real, known_categorical, observed,
           static_tables, known_cat_tables, real_W, real_b, obs_W, obs_b):
    f32, bf16 = jnp.float32, jnp.bfloat16
    nf = N_KR + N_KC
    kw, ow = H * nf, H * N_OBS

    ar_h = jnp.arange(H)
    ar_kr = jnp.arange(N_KR)[:, None]
    ar_kc = jnp.arange(N_KC)[:, None]
    mr = jnp.zeros((N_KR, kw), f32).at[
        jnp.broadcast_to(ar_kr, (N_KR, H)),
        ar_h[None, :] * nf + ar_kr].set(real_W[:, 0, :]).astype(bf16)
    bk = jnp.zeros((kw,), f32).at[(ar_h[None, :] * nf + ar_kr).ravel()].set(
        real_b.ravel()).reshape(1, kw)
    pr = (ar_kc * H + ar_h[None, :]).ravel()
    pc = (ar_h[None, :] * nf + N_KR + ar_kc).ravel()
    p = jnp.zeros((H * N_KC, kw), bf16).at[pr, pc].set(1)
    ar_ob = jnp.arange(N_OBS)[:, None]
    mo = jnp.zeros((N_OBS, ow), f32).at[
        jnp.broadcast_to(ar_ob, (N_OBS, H)),
        ar_h[None, :] * N_OBS + ar_ob].set(obs_W[:, 0, :]).astype(bf16)
    bo = jnp.zeros((ow,), f32).at[(ar_h[None, :] * N_OBS + ar_ob).ravel()].set(
        obs_b.ravel()).reshape(1, ow)

    static_rows, g = _sc_gather(
        static.reshape(B * N_STATIC),
        known_categorical.reshape(N * N_KC),
        static_tables.reshape(N_STATIC * STATIC_VOCAB, H),
        known_cat_tables.reshape(N_KC * KC_VOCAB, H),
    )

    known_flat, obs_flat = _tc_assemble(
        known_real.reshape(N, N_KR),
        observed.reshape(N, N_OBS),
        g.reshape(N, H * N_KC),
        mr, mo, p, bk, bo,
    )

    return (
        static_rows.reshape(B, N_STATIC, H),
        known_flat.reshape(B, T, H, nf),
        obs_flat.reshape(B, T, H, N_OBS),
    )


# speedup vs baseline: 1.2825x; 1.2825x over previous
import functools

import jax
import jax.numpy as jnp
from jax import lax
from jax.experimental import pallas as pl
from jax.experimental.pallas import tpu as pltpu
from jax.experimental.pallas import tpu_sc as plsc

B, T, H = 1024, 200, 32
N_STATIC, N_KC, N_KR, N_OBS = 3, 3, 4, 3
STATIC_VOCAB, KC_VOCAB = 100000, 1000
N = B * T

NC, NS = 2, 16
NW = NC * NS

S_PER_W = (B * N_STATIC) // NW
C_PER_W = (N * N_KC) // NW
STREAM_ROWS = 128
STREAMS_PER_CHUNK = 10
CHUNK = STREAM_ROWS * STREAMS_PER_CHUNK
N_CHUNKS = C_PER_W // CHUNK


def _sc_gather(static_flat, kc_flat, st_tab, kc_tab):
    mesh = plsc.VectorSubcoreMesh(
        core_axis_name="c", subcore_axis_name="s", num_cores=NC, num_subcores=NS
    )

    def body(static_hbm, kc_hbm, sttab_hbm, kctab_hbm, sout_hbm, g_hbm,
             sidx, srows, idx_raw, idx2, rows, sem):
        wid = lax.axis_index("s") * NC + lax.axis_index("c")
        iota = lax.iota(jnp.int32, 16)

        sbase = wid * S_PER_W
        pltpu.sync_copy(static_hbm.at[pl.ds(sbase, S_PER_W)], sidx)
        for v in range(S_PER_W // 16):
            k = sbase + v * 16 + iota
            sidx[pl.ds(v * 16, 16)] = (
                sidx[pl.ds(v * 16, 16)] + (k % N_STATIC) * STATIC_VOCAB
            )
        pltpu.async_copy(sttab_hbm.at[sidx], srows, sem).wait()
        pltpu.sync_copy(srows, sout_hbm.at[pl.ds(sbase, S_PER_W)])

        def chunk_body(c, carry):
            base = wid * C_PER_W + c * CHUNK
            pltpu.sync_copy(kc_hbm.at[pl.ds(base, CHUNK)], idx_raw)
            for v in range(CHUNK // 16):
                k = base + v * 16 + iota
                s, col = v // 8, (v % 8) * 16
                idx2[s, pl.ds(col, 16)] = (
                    idx_raw[pl.ds(v * 16, 16)] + (k % N_KC) * KC_VOCAB
                )
            copies = [
                pltpu.async_copy(
                    kctab_hbm.at[idx2.at[s]],
                    rows.at[pl.ds(s * STREAM_ROWS, STREAM_ROWS)],
                    sem,
                )
                for s in range(STREAMS_PER_CHUNK)
            ]
            for cp in copies:
                cp.wait()
            pltpu.sync_copy(rows, g_hbm.at[pl.ds(base, CHUNK)])
            return carry

        lax.fori_loop(0, N_CHUNKS, chunk_body, 0)

    fn = pl.kernel(
        body,
        out_type=[
            jax.ShapeDtypeStruct((B * N_STATIC, H), jnp.float32),
            jax.ShapeDtypeStruct((N * N_KC, H), jnp.float32),
        ],
        mesh=mesh,
        scratch_types=[
            pltpu.VMEM((S_PER_W,), jnp.int32),
            pltpu.VMEM((S_PER_W, H), jnp.float32),
            pltpu.VMEM((CHUNK,), jnp.int32),
            pltpu.VMEM((STREAMS_PER_CHUNK, STREAM_ROWS), jnp.int32),
            pltpu.VMEM((CHUNK, H), jnp.float32),
            pltpu.SemaphoreType.DMA,
        ],
        compiler_params=pltpu.CompilerParams(use_tc_tiling_on_sc=False),
    )
    return fn(static_flat, kc_flat, st_tab, kc_tab)


NB = 2048


def _tc_body(kr_ref, ob_ref, g_ref, mr_ref, mo_ref, p_ref, bk_ref, bo_ref,
             known_ref, obs_ref):
    g = g_ref[...].astype(jnp.bfloat16)
    kr = kr_ref[...].astype(jnp.bfloat16)
    ob = ob_ref[...].astype(jnp.bfloat16)
    dn = (((1,), (0,)), ((), ()))
    known = lax.dot_general(g, p_ref[...], dn, preferred_element_type=jnp.float32)
    known = known + lax.dot_general(
        kr, mr_ref[...], dn, preferred_element_type=jnp.float32
    )
    known_ref[...] = known + bk_ref[...]
    obs = lax.dot_general(ob, mo_ref[...], dn, preferred_element_type=jnp.float32)
    obs_ref[...] = obs + bo_ref[...]


def _tc_assemble(kr2, ob2, g2, mr, mo, p, bk, bo):
    grid = (N // NB,)
    kw = H * (N_KR + N_KC)
    ow = H * N_OBS
    return pl.pallas_call(
        _tc_body,
        grid=grid,
        in_specs=[
            pl.BlockSpec((NB, N_KR), lambda i: (i, 0)),
            pl.BlockSpec((NB, N_OBS), lambda i: (i, 0)),
            pl.BlockSpec((NB, H * N_KC), lambda i: (i, 0)),
            pl.BlockSpec((N_KR, kw), lambda i: (0, 0)),
            pl.BlockSpec((N_OBS, ow), lambda i: (0, 0)),
            pl.BlockSpec((H * N_KC, kw), lambda i: (0, 0)),
            pl.BlockSpec((1, kw), lambda i: (0, 0)),
            pl.BlockSpec((1, ow), lambda i: (0, 0)),
        ],
        out_specs=[
            pl.BlockSpec((NB, kw), lambda i: (i, 0)),
            pl.BlockSpec((NB, ow), lambda i: (i, 0)),
        ],
        out_shape=[
            jax.ShapeDtypeStruct((N, kw), jnp.float32),
            jax.ShapeDtypeStruct((N, ow), jnp.float32),
        ],
    )(kr2, ob2, g2, mr, mo, p, bk, bo)


def kernel(static, known_real, known_categorical, observed,
           static_tables, known_cat_tables, real_W, real_b, obs_W, obs_b):
    f32, bf16 = jnp.float32, jnp.bfloat16
    nf = N_KR + N_KC
    kw, ow = H * nf, H * N_OBS

    ar_h = jnp.arange(H)
    ar_kr = jnp.arange(N_KR)[:, None]
    ar_kc = jnp.arange(N_KC)[:, None]
    mr = jnp.zeros((N_KR, kw), f32).at[
        jnp.broadcast_to(ar_kr, (N_KR, H)),
        ar_h[None, :] * nf + ar_kr].set(real_W[:, 0, :]).astype(bf16)
    bk = jnp.zeros((kw,), f32).at[(ar_h[None, :] * nf + ar_kr).ravel()].set(
        real_b.ravel()).reshape(1, kw)
    pr = (ar_kc * H + ar_h[None, :]).ravel()
    pc = (ar_h[None, :] * nf + N_KR + ar_kc).ravel()
    p = jnp.zeros((H * N_KC, kw), bf16).at[pr, pc].set(1)
    ar_ob = jnp.arange(N_OBS)[:, None]
    mo = jnp.zeros((N_OBS, ow), f32).at[
        jnp.broadcast_to(ar_ob, (N_OBS, H)),
        ar_h[None, :] * N_OBS + ar_ob].set(obs_W[:, 0, :]).astype(bf16)
    bo = jnp.zeros((ow,), f32).at[(ar_h[None, :] * N_OBS + ar_ob).ravel()].set(
        obs_b.ravel()).reshape(1, ow)

    static_rows, g = _sc_gather(
        static.reshape(B * N_STATIC),
        known_categorical.reshape(N * N_KC),
        static_tables.reshape(N_STATIC * STATIC_VOCAB, H),
        known_cat_tables.reshape(N_KC * KC_VOCAB, H),
    )

    known_flat, obs_flat = _tc_assemble(
        known_real.reshape(N, N_KR),
        observed.reshape(N, N_OBS),
        g.reshape(N, H * N_KC),
        mr, mo, p, bk, bo,
    )

    return (
        static_rows.reshape(B, N_STATIC, H),
        known_flat.reshape(B, T, H, nf),
        obs_flat.reshape(B, T, H, N_OBS),
    )


# speedup vs baseline: 2.9935x; 2.3342x over previous
import jax
import jax.numpy as jnp
from jax import lax
from jax.experimental import pallas as pl
from jax.experimental.pallas import tpu as pltpu
from jax.experimental.pallas import tpu_sc as plsc

B, T, H = 1024, 200, 32
N_STATIC, N_KC, N_KR, N_OBS = 3, 3, 4, 3
STATIC_VOCAB, KC_VOCAB = 100000, 1000
N = B * T
NF = N_KR + N_KC
KW, OW = H * NF, H * N_OBS
DW = H * N_KR

NC, NS = 2, 16
NW = NC * NS

PAIR_TILE = H * B
MAX_I = (T + NW - 1) // NW
PAIRS_PER_W = (N_STATIC * H) // NW


def _sc_gather(static_flat, kc_flat, st_words, kc_tab):
    mesh = plsc.VectorSubcoreMesh(
        core_axis_name="c", subcore_axis_name="s", num_cores=NC, num_subcores=NS
    )

    def body(static_hbm, kc_hbm, stw_hbm, kctab_hbm, sout_hbm, g_hbm,
             sidx, swords, idxb, rows2d, rowst, sem):
        wid = lax.axis_index("s") * NC + lax.axis_index("c")
        iota = lax.iota(jnp.int32, 16)

        for p in range(PAIRS_PER_W):
            pid = wid * PAIRS_PER_W + p
            j = pid // H
            h = pid - j * H
            pltpu.sync_copy(static_hbm.at[pl.ds(j * B, B)], sidx)
            off = j * (H * STATIC_VOCAB) + h * STATIC_VOCAB
            for v in range(B // 16):
                sidx[pl.ds(v * 16, 16)] = sidx[pl.ds(v * 16, 16)] + off
            copies = [
                pltpu.async_copy(
                    stw_hbm.at[sidx.at[pl.ds(s * 128, 128)]],
                    swords.at[pl.ds(s * 128, 128)],
                    sem,
                )
                for s in range(B // 128)
            ]
            for cp in copies:
                cp.wait()
            pltpu.sync_copy(swords, sout_hbm.at[pl.ds(pid * B, B)])

        for j in range(N_KC):
            def pair_body(i, carry, j=j):
                t = wid + NW * i

                @pl.when(t < T)
                def _():
                    base_in = j * N + t * B
                    pltpu.sync_copy(kc_hbm.at[pl.ds(base_in, B)], idxb)
                    for v in range(B // 16):
                        idxb[pl.ds(v * 16, 16)] = (
                            idxb[pl.ds(v * 16, 16)] + j * KC_VOCAB
                        )
                    copies = [
                        pltpu.async_copy(
                            kctab_hbm.at[idxb.at[pl.ds(s * 128, 128)]],
                            rows2d.at[pl.ds(s * 128, 128)],
                            sem,
                        )
                        for s in range(B // 128)
                    ]
                    for cp in copies:
                        cp.wait()

                    def h_body(h, c2):
                        cvec = jnp.broadcast_to(h, (16,))
                        d0 = ((h // 8) * (8 * 8 * 128)) + (h % 8) * 128
                        for cb in range(8):
                            for cv in range(8):
                                rvec = iota + (cb * 128 + cv * 16)
                                vals = plsc.load_gather(rows2d, [rvec, cvec])
                                rowst[pl.ds(d0 + cb * 1024 + cv * 16, 16)] = vals
                        return c2

                    lax.fori_loop(0, H, h_body, 0)
                    pltpu.sync_copy(
                        rowst,
                        g_hbm.at[pl.ds((j * T + t) * PAIR_TILE, PAIR_TILE)],
                    )

                return carry

            lax.fori_loop(0, MAX_I, pair_body, 0)

    fn = pl.kernel(
        body,
        out_type=[
            jax.ShapeDtypeStruct((N_STATIC * H * B,), jnp.float32),
            jax.ShapeDtypeStruct((N_KC * T * PAIR_TILE,), jnp.float32),
        ],
        mesh=mesh,
        scratch_types=[
            pltpu.VMEM((B,), jnp.int32),
            pltpu.VMEM((B,), jnp.float32),
            pltpu.VMEM((B,), jnp.int32),
            pltpu.VMEM((B, H), jnp.float32),
            pltpu.VMEM((PAIR_TILE,), jnp.float32),
            pltpu.SemaphoreType.DMA,
        ],
        compiler_params=pltpu.CompilerParams(
            use_tc_tiling_on_sc=False, needs_layout_passes=False),
    )
    return fn(static_flat, kc_flat, st_words, kc_tab)


TB = 8


def _tc_body(kr_ref, ob_ref, g_ref, mrt_ref, mot_ref, bk_ref, bo_ref,
             known_ref, obs_ref):
    f32 = jnp.float32
    dn = (((1,), (0,)), ((), ()))
    for t in range(TB):
        dense = lax.dot_general(mrt_ref[...], kr_ref[t], dn,
                                preferred_element_type=f32)
        known_ref[t, pl.ds(0, DW)] = dense + bk_ref[...]
        for j in range(N_KC):
            known_ref[t, pl.ds(DW + j * H, H)] = g_ref[j, t]
        obs = lax.dot_general(mot_ref[...], ob_ref[:, t], dn,
                              preferred_element_type=f32)
        obs_ref[t] = obs + bo_ref[...]


def _tc_assemble(kr_t, ob_t, g4, mrt, mot, bk, bo):
    return pl.pallas_call(
        _tc_body,
        grid=(T // TB,),
        in_specs=[
            pl.BlockSpec((TB, N_KR, B), lambda i: (i, 0, 0)),
            pl.BlockSpec((N_OBS, TB, B), lambda i: (0, i, 0)),
            pl.BlockSpec((N_KC, TB, H, B), lambda i: (0, i, 0, 0)),
            pl.BlockSpec((DW, N_KR), lambda i: (0, 0)),
            pl.BlockSpec((OW, N_OBS), lambda i: (0, 0)),
            pl.BlockSpec((DW, 1), lambda i: (0, 0)),
            pl.BlockSpec((OW, 1), lambda i: (0, 0)),
        ],
        out_specs=[
            pl.BlockSpec((TB, KW, B), lambda i: (i, 0, 0)),
            pl.BlockSpec((TB, OW, B), lambda i: (i, 0, 0)),
        ],
        out_shape=[
            jax.ShapeDtypeStruct((T, KW, B), jnp.float32),
            jax.ShapeDtypeStruct((T, OW, B), jnp.float32),
        ],
    )(kr_t, ob_t, g4, mrt, mot, bk, bo)


def kernel(static, known_real, known_categorical, observed,
           static_tables, known_cat_tables, real_W, real_b, obs_W, obs_b):
    f32 = jnp.float32

    mrt = jnp.zeros((DW, N_KR), f32).at[
        jnp.arange(DW), jnp.arange(DW) // H].set(real_W[:, 0, :].ravel())
    bk = real_b.reshape(DW, 1)
    mot = jnp.zeros((OW, N_OBS), f32).at[
        jnp.arange(OW), jnp.arange(OW) // H].set(obs_W[:, 0, :].ravel())
    bo = obs_b.reshape(OW, 1)

    static_flat = jnp.transpose(static).reshape(N_STATIC * B)
    kc_flat = jnp.transpose(known_categorical, (2, 1, 0)).reshape(N_KC * N)
    st_words = jnp.transpose(static_tables, (0, 2, 1)).reshape(-1)
    kc_tab = known_cat_tables.reshape(N_KC * KC_VOCAB, H)
    kr_t = jnp.transpose(known_real, (1, 2, 0))
    ob_t = jnp.transpose(observed, (2, 1, 0))

    static_words, g = _sc_gather(static_flat, kc_flat, st_words, kc_tab)

    g6 = g.reshape(N_KC, T, H // 8, B // 128, 8, 128)
    g4 = jnp.transpose(g6, (0, 1, 2, 4, 3, 5)).reshape(N_KC, T, H, B)

    known_t, obs_t = _tc_assemble(kr_t, ob_t, g4, mrt, mot, bk, bo)

    return (
        jnp.transpose(static_words.reshape(N_STATIC, H, B), (2, 0, 1)),
        jnp.transpose(known_t.reshape(T, NF, H, B), (3, 0, 2, 1)),
        jnp.transpose(obs_t.reshape(T, N_OBS, H, B), (3, 0, 2, 1)),
    )


# speedup vs baseline: 6.9203x; 2.3118x over previous
import jax
import jax.numpy as jnp
from jax import lax
from jax.experimental import pallas as pl
from jax.experimental.pallas import tpu as pltpu
from jax.experimental.pallas import tpu_sc as plsc

B, T, H = 1024, 200, 32
N_STATIC, N_KC, N_KR, N_OBS = 3, 3, 4, 3
STATIC_VOCAB, KC_VOCAB = 100000, 1000
N = B * T
NF = N_KR + N_KC
KW, OW = H * NF, H * N_OBS
DW = H * N_KR

NC, NS = 2, 16
NW = NC * NS

PAIR_TILE = H * B
MAX_I = (T + NW - 1) // NW
PAIRS_PER_W = (N_STATIC * H) // NW


def _sc_gather(static_flat, kc_flat, st_words, kc_tab):
    mesh = plsc.VectorSubcoreMesh(
        core_axis_name="c", subcore_axis_name="s", num_cores=NC, num_subcores=NS
    )

    def body(static_hbm, kc_hbm, stw_hbm, kctab_hbm, sout_hbm, g_hbm,
             sidx, swords, idxb, idxp, rows2d, sem):
        wid = lax.axis_index("s") * NC + lax.axis_index("c")
        iota = lax.iota(jnp.int32, 16)

        for p in range(PAIRS_PER_W):
            pid = wid * PAIRS_PER_W + p
            j = pid // H
            h = pid - j * H
            pltpu.sync_copy(static_hbm.at[pl.ds(j * B, B)], sidx)
            off = j * (H * STATIC_VOCAB) + h * STATIC_VOCAB
            for v in range(B // 16):
                sidx[pl.ds(v * 16, 16)] = sidx[pl.ds(v * 16, 16)] + off
            copies = [
                pltpu.async_copy(
                    stw_hbm.at[sidx.at[pl.ds(s * 128, 128)]],
                    swords.at[pl.ds(s * 128, 128)],
                    sem,
                )
                for s in range(B // 128)
            ]
            for cp in copies:
                cp.wait()
            pltpu.sync_copy(swords, sout_hbm.at[pl.ds(pid * B, B)])

        for j in range(N_KC):
            def pair_body(i, carry, j=j):
                t = wid + NW * i

                @pl.when(t < T)
                def _():
                    base_in = j * N + t * B
                    pltpu.sync_copy(kc_hbm.at[pl.ds(base_in, B)], idxb)
                    for v in range(B // 16):
                        r = iota + v * 16
                        bv = (r & 3) * 256 + (r >> 2)
                        idxp[pl.ds(v * 16, 16)] = (
                            plsc.load_gather(idxb, [bv]) + j * KC_VOCAB
                        )
                    copies = [
                        pltpu.async_copy(
                            kctab_hbm.at[idxp.at[pl.ds(s * 128, 128)]],
                            rows2d.at[pl.ds(s * 128, 128)],
                            sem,
                        )
                        for s in range(B // 128)
                    ]
                    for cp in copies:
                        cp.wait()
                    pltpu.sync_copy(rows2d, g_hbm.at[pl.ds(base_in, B)])

                return carry

            lax.fori_loop(0, MAX_I, pair_body, 0)

    fn = pl.kernel(
        body,
        out_type=[
            jax.ShapeDtypeStruct((N_STATIC * H * B,), jnp.float32),
            jax.ShapeDtypeStruct((N_KC * N, H), jnp.float32),
        ],
        mesh=mesh,
        scratch_types=[
            pltpu.VMEM((B,), jnp.int32),
            pltpu.VMEM((B,), jnp.float32),
            pltpu.VMEM((B,), jnp.int32),
            pltpu.VMEM((B,), jnp.int32),
            pltpu.VMEM((B, H), jnp.float32),
            pltpu.SemaphoreType.DMA,
        ],
        compiler_params=pltpu.CompilerParams(
            use_tc_tiling_on_sc=False, needs_layout_passes=False),
    )
    return fn(static_flat, kc_flat, st_words, kc_tab)


TB = 8


def _tc_body(kr_ref, ob_ref, g_ref, mrt_ref, mot_ref, bk_ref, bo_ref,
             known_ref, obs_ref):
    f32 = jnp.float32
    dn = (((1,), (0,)), ((), ()))
    for t in range(TB):
        dense = lax.dot_general(mrt_ref[...], kr_ref[t], dn,
                                preferred_element_type=f32)
        known_ref[t, pl.ds(0, DW)] = dense + bk_ref[...]
        for j in range(N_KC):
            gt = jnp.swapaxes(g_ref[j, t], 0, 1)
            for q in range(4):
                known_ref[t, pl.ds(DW + j * H, H), pl.ds(q * 256, 256)] = (
                    gt[q * H:(q + 1) * H]
                )
        obs = lax.dot_general(mot_ref[...], ob_ref[:, t], dn,
                              preferred_element_type=f32)
        obs_ref[t] = obs + bo_ref[...]


def _tc_assemble(kr_t, ob_t, g4, mrt, mot, bk, bo):
    return pl.pallas_call(
        _tc_body,
        grid=(T // TB,),
        in_specs=[
            pl.BlockSpec((TB, N_KR, B), lambda i: (i, 0, 0)),
            pl.BlockSpec((N_OBS, TB, B), lambda i: (0, i, 0)),
            pl.BlockSpec((N_KC, TB, B // 4, 128), lambda i: (0, i, 0, 0)),
            pl.BlockSpec((DW, N_KR), lambda i: (0, 0)),
            pl.BlockSpec((OW, N_OBS), lambda i: (0, 0)),
            pl.BlockSpec((DW, 1), lambda i: (0, 0)),
            pl.BlockSpec((OW, 1), lambda i: (0, 0)),
        ],
        out_specs=[
            pl.BlockSpec((TB, KW, B), lambda i: (i, 0, 0)),
            pl.BlockSpec((TB, OW, B), lambda i: (i, 0, 0)),
        ],
        out_shape=[
            jax.ShapeDtypeStruct((T, KW, B), jnp.float32),
            jax.ShapeDtypeStruct((T, OW, B), jnp.float32),
        ],
    )(kr_t, ob_t, g4, mrt, mot, bk, bo)


def kernel(static, known_real, known_categorical, observed,
           static_tables, known_cat_tables, real_W, real_b, obs_W, obs_b):
    f32 = jnp.float32

    mrt = jnp.zeros((DW, N_KR), f32).at[
        jnp.arange(DW), jnp.arange(DW) // H].set(real_W[:, 0, :].ravel())
    bk = real_b.reshape(DW, 1)
    mot = jnp.zeros((OW, N_OBS), f32).at[
        jnp.arange(OW), jnp.arange(OW) // H].set(obs_W[:, 0, :].ravel())
    bo = obs_b.reshape(OW, 1)

    static_flat = jnp.transpose(static).reshape(N_STATIC * B)
    kc_flat = jnp.transpose(known_categorical, (2, 1, 0)).reshape(N_KC * N)
    st_words = jnp.transpose(static_tables, (0, 2, 1)).reshape(-1)
    kc_tab = known_cat_tables.reshape(N_KC * KC_VOCAB, H)
    kr_t = jnp.transpose(known_real, (1, 2, 0))
    ob_t = jnp.transpose(observed, (2, 1, 0))

    static_words, g = _sc_gather(static_flat, kc_flat, st_words, kc_tab)

    g4 = g.reshape(N_KC, T, B // 4, 128)

    known_t, obs_t = _tc_assemble(kr_t, ob_t, g4, mrt, mot, bk, bo)

    return (
        jnp.transpose(static_words.reshape(N_STATIC, H, B), (2, 0, 1)),
        jnp.transpose(known_t.reshape(T, NF, H, B), (3, 0, 2, 1)),
        jnp.transpose(obs_t.reshape(T, N_OBS, H, B), (3, 0, 2, 1)),
    )


# speedup vs baseline: 7.0127x; 1.0134x over previous
import jax
import jax.numpy as jnp
from jax import lax
from jax.experimental import pallas as pl
from jax.experimental.pallas import tpu as pltpu
from jax.experimental.pallas import tpu_sc as plsc

B, T, H = 1024, 200, 32
N_STATIC, N_KC, N_KR, N_OBS = 3, 3, 4, 3
STATIC_VOCAB, KC_VOCAB = 100000, 1000
N = B * T
NF = N_KR + N_KC
KW, OW = H * NF, H * N_OBS
DW = H * N_KR

NC, NS = 2, 16
NW = NC * NS

PAIR_TILE = H * B
MAX_I = (T + NW - 1) // NW
PAIRS_PER_W = (N_STATIC * H) // NW


def _sc_gather(static_flat, kc_flat, st_words, kc_tab):
    mesh = plsc.VectorSubcoreMesh(
        core_axis_name="c", subcore_axis_name="s", num_cores=NC, num_subcores=NS
    )

    def body(static_hbm, kc_hbm, stw_hbm, kctab_hbm, sout_hbm, g_hbm,
             sidx, swords, idxb, idxp, rows2d_a, rows2d_b, sem, osem_a, osem_b):
        wid = lax.axis_index("s") * NC + lax.axis_index("c")
        iota = lax.iota(jnp.int32, 16)

        for p in range(PAIRS_PER_W):
            pid = wid * PAIRS_PER_W + p
            j = pid // H
            h = pid - j * H
            pltpu.sync_copy(static_hbm.at[pl.ds(j * B, B)], sidx)
            off = j * (H * STATIC_VOCAB) + h * STATIC_VOCAB
            for v in range(B // 16):
                sidx[pl.ds(v * 16, 16)] = sidx[pl.ds(v * 16, 16)] + off
            copies = [
                pltpu.async_copy(
                    stw_hbm.at[sidx.at[pl.ds(s * 128, 128)]],
                    swords.at[pl.ds(s * 128, 128)],
                    sem,
                )
                for s in range(B // 128)
            ]
            for cp in copies:
                cp.wait()
            pltpu.sync_copy(swords, sout_hbm.at[pl.ds(pid * B, B)])

        rows_bufs = (rows2d_a, rows2d_b)
        osems = (osem_a, osem_b)
        for j in range(N_KC):
            def k_body(k, carry, j=j):
                for par in range(2):
                    i = 2 * k + par
                    t = wid + NW * i

                    @pl.when(t < T)
                    def _(par=par, i=i, t=t):
                        rows = rows_bufs[par]
                        osem = osems[par]
                        base_in = j * N + t * B
                        pltpu.sync_copy(kc_hbm.at[pl.ds(base_in, B)], idxb)
                        for v in range(B // 16):
                            r = iota + v * 16
                            bv = (r & 3) * 256 + (r >> 2)
                            idxp[pl.ds(v * 16, 16)] = (
                                plsc.load_gather(idxb, [bv]) + j * KC_VOCAB
                            )
                        @pl.when(k > 0)
                        def _():
                            pltpu.make_async_copy(
                                rows, g_hbm.at[pl.ds(0, B)], osem).wait()
                        copies = [
                            pltpu.async_copy(
                                kctab_hbm.at[idxp.at[pl.ds(s * 128, 128)]],
                                rows.at[pl.ds(s * 128, 128)],
                                sem,
                            )
                            for s in range(B // 128)
                        ]
                        for cp in copies:
                            cp.wait()
                        pltpu.async_copy(
                            rows, g_hbm.at[pl.ds(base_in, B)], osem)

                return carry

            lax.fori_loop(0, (MAX_I + 1) // 2, k_body, 0)
            for par in range(2):
                pltpu.make_async_copy(
                    rows_bufs[par], g_hbm.at[pl.ds(0, B)], osems[par]).wait()

    fn = pl.kernel(
        body,
        out_type=[
            jax.ShapeDtypeStruct((N_STATIC * H * B,), jnp.float32),
            jax.ShapeDtypeStruct((N_KC * N, H), jnp.float32),
        ],
        mesh=mesh,
        scratch_types=[
            pltpu.VMEM((B,), jnp.int32),
            pltpu.VMEM((B,), jnp.float32),
            pltpu.VMEM((B,), jnp.int32),
            pltpu.VMEM((B,), jnp.int32),
            pltpu.VMEM((B, H), jnp.float32),
            pltpu.VMEM((B, H), jnp.float32),
            pltpu.SemaphoreType.DMA,
            pltpu.SemaphoreType.DMA,
            pltpu.SemaphoreType.DMA,
        ],
        compiler_params=pltpu.CompilerParams(
            use_tc_tiling_on_sc=False, needs_layout_passes=False),
    )
    return fn(static_flat, kc_flat, st_words, kc_tab)


TB = 10
TBO = 40


def _tc_known_body(kr_ref, g_ref, mrt_ref, bk_ref, known_ref):
    f32 = jnp.float32
    dn = (((1,), (0,)), ((), ()))
    for t in range(TB):
        dense = lax.dot_general(mrt_ref[...], kr_ref[t], dn,
                                preferred_element_type=f32)
        known_ref[t, pl.ds(0, DW)] = dense + bk_ref[...]
        for j in range(N_KC):
            gt = jnp.swapaxes(g_ref[j, t], 0, 1)
            for q in range(4):
                known_ref[t, pl.ds(DW + j * H, H), pl.ds(q * 256, 256)] = (
                    gt[q * H:(q + 1) * H]
                )


def _tc_obs_body(ob_ref, mot_ref, bo_ref, obs_ref):
    f32 = jnp.float32
    dn = (((1,), (0,)), ((), ()))
    for t in range(TBO):
        obs = lax.dot_general(mot_ref[...], ob_ref[:, t], dn,
                              preferred_element_type=f32)
        obs_ref[t] = obs + bo_ref[...]


def _tc_assemble(kr_t, ob_t, g4, mrt, mot, bk, bo):
    known_t = pl.pallas_call(
        _tc_known_body,
        grid=(T // TB,),
        in_specs=[
            pl.BlockSpec((TB, N_KR, B), lambda i: (i, 0, 0)),
            pl.BlockSpec((N_KC, TB, B // 4, 128), lambda i: (0, i, 0, 0)),
            pl.BlockSpec((DW, N_KR), lambda i: (0, 0)),
            pl.BlockSpec((DW, 1), lambda i: (0, 0)),
        ],
        out_specs=[pl.BlockSpec((TB, KW, B), lambda i: (i, 0, 0))],
        out_shape=[jax.ShapeDtypeStruct((T, KW, B), jnp.float32)],
    )(kr_t, g4, mrt, bk)[0]
    obs_t = pl.pallas_call(
        _tc_obs_body,
        grid=(T // TBO,),
        in_specs=[
            pl.BlockSpec((N_OBS, TBO, B), lambda i: (0, i, 0)),
            pl.BlockSpec((OW, N_OBS), lambda i: (0, 0)),
            pl.BlockSpec((OW, 1), lambda i: (0, 0)),
        ],
        out_specs=[pl.BlockSpec((TBO, OW, B), lambda i: (i, 0, 0))],
        out_shape=[jax.ShapeDtypeStruct((T, OW, B), jnp.float32)],
    )(ob_t, mot, bo)[0]
    return known_t, obs_t


def kernel(static, known_real, known_categorical, observed,
           static_tables, known_cat_tables, real_W, real_b, obs_W, obs_b):
    f32 = jnp.float32

    mrt = jnp.zeros((DW, N_KR), f32).at[
        jnp.arange(DW), jnp.arange(DW) // H].set(real_W[:, 0, :].ravel())
    bk = real_b.reshape(DW, 1)
    mot = jnp.zeros((OW, N_OBS), f32).at[
        jnp.arange(OW), jnp.arange(OW) // H].set(obs_W[:, 0, :].ravel())
    bo = obs_b.reshape(OW, 1)

    static_flat = jnp.transpose(static).reshape(N_STATIC * B)
    kc_flat = jnp.transpose(known_categorical, (2, 1, 0)).reshape(N_KC * N)
    st_words = jnp.transpose(static_tables, (0, 2, 1)).reshape(-1)
    kc_tab = known_cat_tables.reshape(N_KC * KC_VOCAB, H)
    kr_t = jnp.transpose(known_real, (1, 2, 0))
    ob_t = jnp.transpose(observed, (2, 1, 0))

    static_words, g = _sc_gather(static_flat, kc_flat, st_words, kc_tab)

    g4 = g.reshape(N_KC, T, B // 4, 128)

    known_t, obs_t = _tc_assemble(kr_t, ob_t, g4, mrt, mot, bk, bo)

    return (
        jnp.transpose(static_words.reshape(N_STATIC, H, B), (2, 0, 1)),
        jnp.transpose(known_t.reshape(T, NF, H, B), (3, 0, 2, 1)),
        jnp.transpose(obs_t.reshape(T, N_OBS, H, B), (3, 0, 2, 1)),
    )


# speedup vs baseline: 7.0191x; 1.0009x over previous
import jax
import jax.numpy as jnp
from jax import lax
from jax.experimental import pallas as pl
from jax.experimental.pallas import tpu as pltpu
from jax.experimental.pallas import tpu_sc as plsc

B, T, H = 1024, 200, 32
N_STATIC, N_KC, N_KR, N_OBS = 3, 3, 4, 3
STATIC_VOCAB, KC_VOCAB = 100000, 1000
N = B * T
NF = N_KR + N_KC
KW, OW = H * NF, H * N_OBS
DW = H * N_KR

NC, NS = 2, 16
NW = NC * NS

PAIR_TILE = H * B
MAX_I = (T + NW - 1) // NW
PAIRS_PER_W = (N_STATIC * H) // NW


def _sc_gather(static_flat, kc_flat, st_words, kc_tab):
    mesh = plsc.VectorSubcoreMesh(
        core_axis_name="c", subcore_axis_name="s", num_cores=NC, num_subcores=NS
    )

    def body(static_hbm, kc_hbm, stw_hbm, kctab_hbm, sout_hbm, g_hbm,
             sidx, swords, idxb, idxp, rows2d_a, rows2d_b, sem, osem_a, osem_b):
        wid = lax.axis_index("s") * NC + lax.axis_index("c")
        iota = lax.iota(jnp.int32, 16)

        for p in range(PAIRS_PER_W):
            pid = wid * PAIRS_PER_W + p
            j = pid // H
            h = pid - j * H
            pltpu.sync_copy(static_hbm.at[pl.ds(j * B, B)], sidx)
            off = j * (H * STATIC_VOCAB) + h * STATIC_VOCAB
            for v in range(B // 16):
                sidx[pl.ds(v * 16, 16)] = sidx[pl.ds(v * 16, 16)] + off
            copies = [
                pltpu.async_copy(
                    stw_hbm.at[sidx.at[pl.ds(s * 128, 128)]],
                    swords.at[pl.ds(s * 128, 128)],
                    sem,
                )
                for s in range(B // 128)
            ]
            for cp in copies:
                cp.wait()
            pltpu.sync_copy(swords, sout_hbm.at[pl.ds(pid * B, B)])

        rows_bufs = (rows2d_a, rows2d_b)
        osems = (osem_a, osem_b)
        for j in range(N_KC):
            def k_body(k, carry, j=j):
                for par in range(2):
                    i = 2 * k + par
                    t = wid + NW * i

                    @pl.when(t < T)
                    def _(par=par, i=i, t=t):
                        rows = rows_bufs[par]
                        osem = osems[par]
                        base_in = j * N + t * B
                        pltpu.sync_copy(kc_hbm.at[pl.ds(base_in, B)], idxb)
                        for v in range(B // 16):
                            r = iota + v * 16
                            bv = (r & 3) * 256 + (r >> 2)
                            idxp[pl.ds(v * 16, 16)] = (
                                plsc.load_gather(idxb, [bv]) + j * KC_VOCAB
                            )
                        @pl.when(k > 0)
                        def _():
                            pltpu.make_async_copy(
                                rows, g_hbm.at[pl.ds(0, B)], osem).wait()
                        copies = [
                            pltpu.async_copy(
                                kctab_hbm.at[idxp.at[pl.ds(s * 128, 128)]],
                                rows.at[pl.ds(s * 128, 128)],
                                sem,
                            )
                            for s in range(B // 128)
                        ]
                        for cp in copies:
                            cp.wait()
                        pltpu.async_copy(
                            rows, g_hbm.at[pl.ds(base_in, B)], osem)

                return carry

            lax.fori_loop(0, (MAX_I + 1) // 2, k_body, 0)
            for par in range(2):
                pltpu.make_async_copy(
                    rows_bufs[par], g_hbm.at[pl.ds(0, B)], osems[par]).wait()

    fn = pl.kernel(
        body,
        out_type=[
            jax.ShapeDtypeStruct((N_STATIC * H * B,), jnp.float32),
            jax.ShapeDtypeStruct((N_KC * N, H), jnp.float32),
        ],
        mesh=mesh,
        scratch_types=[
            pltpu.VMEM((B,), jnp.int32),
            pltpu.VMEM((B,), jnp.float32),
            pltpu.VMEM((B,), jnp.int32),
            pltpu.VMEM((B,), jnp.int32),
            pltpu.VMEM((B, H), jnp.float32),
            pltpu.VMEM((B, H), jnp.float32),
            pltpu.SemaphoreType.DMA,
            pltpu.SemaphoreType.DMA,
            pltpu.SemaphoreType.DMA,
        ],
        compiler_params=pltpu.CompilerParams(
            use_tc_tiling_on_sc=False, needs_layout_passes=False),
    )
    return fn(static_flat, kc_flat, st_words, kc_tab)


TB = 20
TBO = 40


def _tc_known_body(kr_ref, g_ref, mrt_ref, bk_ref, known_ref):
    f32 = jnp.float32
    dn = (((1,), (0,)), ((), ()))
    for t in range(TB):
        dense = lax.dot_general(mrt_ref[...], kr_ref[t], dn,
                                preferred_element_type=f32)
        known_ref[t, pl.ds(0, DW)] = dense + bk_ref[...]
        for j in range(N_KC):
            gt = jnp.swapaxes(g_ref[j, t], 0, 1)
            for q in range(4):
                known_ref[t, pl.ds(DW + j * H, H), pl.ds(q * 256, 256)] = (
                    gt[q * H:(q + 1) * H]
                )


def _tc_obs_body(ob_ref, mot_ref, bo_ref, obs_ref):
    f32 = jnp.float32
    dn = (((1,), (0,)), ((), ()))
    for t in range(TBO):
        obs = lax.dot_general(mot_ref[...], ob_ref[:, t], dn,
                              preferred_element_type=f32)
        obs_ref[t] = obs + bo_ref[...]


def _tc_assemble(kr_t, ob_t, g4, mrt, mot, bk, bo):
    known_t = pl.pallas_call(
        _tc_known_body,
        grid=(T // TB,),
        in_specs=[
            pl.BlockSpec((TB, N_KR, B), lambda i: (i, 0, 0)),
            pl.BlockSpec((N_KC, TB, B // 4, 128), lambda i: (0, i, 0, 0)),
            pl.BlockSpec((DW, N_KR), lambda i: (0, 0)),
            pl.BlockSpec((DW, 1), lambda i: (0, 0)),
        ],
        out_specs=[pl.BlockSpec((TB, KW, B), lambda i: (i, 0, 0))],
        out_shape=[jax.ShapeDtypeStruct((T, KW, B), jnp.float32)],
    )(kr_t, g4, mrt, bk)[0]
    obs_t = pl.pallas_call(
        _tc_obs_body,
        grid=(T // TBO,),
        in_specs=[
            pl.BlockSpec((N_OBS, TBO, B), lambda i: (0, i, 0)),
            pl.BlockSpec((OW, N_OBS), lambda i: (0, 0)),
            pl.BlockSpec((OW, 1), lambda i: (0, 0)),
        ],
        out_specs=[pl.BlockSpec((TBO, OW, B), lambda i: (i, 0, 0))],
        out_shape=[jax.ShapeDtypeStruct((T, OW, B), jnp.float32)],
    )(ob_t, mot, bo)[0]
    return known_t, obs_t


def kernel(static, known_real, known_categorical, observed,
           static_tables, known_cat_tables, real_W, real_b, obs_W, obs_b):
    f32 = jnp.float32

    mrt = jnp.zeros((DW, N_KR), f32).at[
        jnp.arange(DW), jnp.arange(DW) // H].set(real_W[:, 0, :].ravel())
    bk = real_b.reshape(DW, 1)
    mot = jnp.zeros((OW, N_OBS), f32).at[
        jnp.arange(OW), jnp.arange(OW) // H].set(obs_W[:, 0, :].ravel())
    bo = obs_b.reshape(OW, 1)

    static_flat = jnp.transpose(static).reshape(N_STATIC * B)
    kc_flat = jnp.transpose(known_categorical, (2, 1, 0)).reshape(N_KC * N)
    st_words = jnp.transpose(static_tables, (0, 2, 1)).reshape(-1)
    kc_tab = known_cat_tables.reshape(N_KC * KC_VOCAB, H)
    kr_t = jnp.transpose(known_real, (1, 2, 0))
    ob_t = jnp.transpose(observed, (2, 1, 0))

    static_words, g = _sc_gather(static_flat, kc_flat, st_words, kc_tab)

    g4 = g.reshape(N_KC, T, B // 4, 128)

    known_t, obs_t = _tc_assemble(kr_t, ob_t, g4, mrt, mot, bk, bo)

    return (
        jnp.transpose(static_words.reshape(N_STATIC, H, B), (2, 0, 1)),
        jnp.transpose(known_t.reshape(T, NF, H, B), (3, 0, 2, 1)),
        jnp.transpose(obs_t.reshape(T, N_OBS, H, B), (3, 0, 2, 1)),
    )
